# baseline (speedup 1.0000x reference)
import functools

import jax
import jax.numpy as jnp
from jax import lax
from jax.experimental import pallas as pl
from jax.experimental.pallas import tpu as pltpu
from jax.experimental.pallas import tpu_sc as plsc

_SUB = 128
_SUBS_PER_CHUNK = 4
_CHUNK = _SUB * _SUBS_PER_CHUNK


@functools.lru_cache(maxsize=None)
def _build_lookup(n_rows: int, v: int, d: int):
    info = plsc.get_sparse_core_info()
    nc, ns, nl = info.num_cores, info.num_subcores, info.num_lanes
    nw = nc * ns
    assert n_rows % (nw * 2 * _CHUNK) == 0
    chunks_per_w = n_rows // (nw * _CHUNK)
    subs_per_w = chunks_per_w * _SUBS_PER_CHUNK
    mesh = plsc.VectorSubcoreMesh(core_axis_name="c", subcore_axis_name="s")

    @functools.partial(
        pl.kernel,
        mesh=mesh,
        out_type=jax.ShapeDtypeStruct((n_rows, d), jnp.float32),
        scratch_types=[
            pltpu.VMEM_SHARED((ns * v, d), jnp.float32),
            pltpu.VMEM((2, _SUBS_PER_CHUNK, _SUB), jnp.int32),
            pltpu.VMEM((2, _CHUNK, d), jnp.float32),
            pltpu.SemaphoreType.DMA,
            pltpu.SemaphoreType.DMA,
        ],
        compiler_params=pltpu.CompilerParams(use_tc_tiling_on_sc=False),
    )
    def lookup(ids_hbm, table_hbm, out_hbm, table_s, idx_v, rows_v, sem0, sem1):
        sid = lax.axis_index("s")
        wid = sid * nc + lax.axis_index("c")
        pltpu.sync_copy(table_hbm, table_s.at[pl.ds(sid * v, v)])
        plsc.subcore_barrier()
        sems = (sem0, sem1)
        bias = (sid * v).astype(jnp.int32)

        def fetch_ids(ci, b):
            sub_base = wid * subs_per_w + ci * _SUBS_PER_CHUNK
            pltpu.sync_copy(ids_hbm.at[pl.ds(sub_base, _SUBS_PER_CHUNK)],
                            idx_v.at[b])
            for j in range(_SUBS_PER_CHUNK):
                row = idx_v.at[b].at[j]
                for k in range(_SUB // nl):
                    row[pl.ds(k * nl, nl)] = row[pl.ds(k * nl, nl)] + bias

        def fire_gathers(b):
            return [
                pltpu.async_copy(
                    table_s.at[idx_v.at[b].at[j]],
                    rows_v.at[b].at[pl.ds(j * _SUB, _SUB)],
                    sems[b],
                )
                for j in range(_SUBS_PER_CHUNK)
            ]

        def out_copy(ci, b):
            pltpu.sync_copy(rows_v.at[b],
                            out_hbm.at[pl.ds((wid * subs_per_w
                                              + ci * _SUBS_PER_CHUNK) * _SUB,
                                             _CHUNK)])

        fetch_ids(0, 0)
        g0 = fire_gathers(0)

        def pair_body(p, carry):
            c0 = 2 * p
            fetch_ids(c0 + 1, 1)
            g1 = fire_gathers(1)
            for cpy in g0:
                cpy.wait()
            out_copy(c0, 0)

            @pl.when(p < chunks_per_w // 2 - 1)
            def _():
                fetch_ids(c0 + 2, 0)
                fire_gathers(0)

            for cpy in g1:
                cpy.wait()
            out_copy(c0 + 1, 1)
            return carry

        lax.fori_loop(0, chunks_per_w // 2, pair_body, 0)

    return lookup


def kernel(input_ids, W):
    bsz, seq = input_ids.shape
    v, d = W.shape
    n_rows = bsz * seq
    ids2d = input_ids.astype(jnp.int32).reshape(n_rows // _SUB, _SUB)
    out = _build_lookup(n_rows, v, d)(ids2d, W)
    return out.reshape(bsz, seq, d)
